# baseline (speedup 1.0000x reference)
import jax
import jax.numpy as jnp
from jax.experimental import pallas as pl
from jax.experimental.pallas import tpu as pltpu

_DN_T = (((1,), (1,)), ((), ()))

_TB = 512


def _mlp_kernel(x_ref, w1_ref, b1_ref, w2_ref, b2_ref, out_ref):
    h = jax.lax.dot_general(x_ref[...], w1_ref[...], _DN_T,
                            preferred_element_type=jnp.float32)
    h = jnp.maximum(h + b1_ref[...], 0.0)
    o = jax.lax.dot_general(h, w2_ref[...], _DN_T,
                            preferred_element_type=jnp.float32)
    out_ref[...] = o + b2_ref[...]


def kernel(x, w1, b1, w2, b2):
    B, F = x.shape
    H = w1.shape[0]
    O = w2.shape[0]

    b1r = b1.reshape(1, H)
    b2r = b2.reshape(1, O)

    return pl.pallas_call(
        _mlp_kernel,
        out_shape=jax.ShapeDtypeStruct((B, O), jnp.float32),
        grid=(pl.cdiv(B, _TB),),
        in_specs=[
            pl.BlockSpec((_TB, F), lambda i: (i, 0)),
            pl.BlockSpec((H, F), lambda i: (0, 0)),
            pl.BlockSpec((1, H), lambda i: (0, 0)),
            pl.BlockSpec((O, H), lambda i: (0, 0)),
            pl.BlockSpec((1, O), lambda i: (0, 0)),
        ],
        out_specs=pl.BlockSpec((_TB, O), lambda i: (i, 0)),
        compiler_params=pltpu.CompilerParams(
            dimension_semantics=("arbitrary",),
        ),
        cost_estimate=pl.CostEstimate(
            flops=2 * B * (F * H + H * O),
            transcendentals=0,
            bytes_accessed=4 * (B * F + B * O + F * H + H * O),
        ),
    )(x, w1, b1r, w2, b2r)


# speedup vs baseline: 1.0032x; 1.0032x over previous
import jax
import jax.numpy as jnp
from jax.experimental import pallas as pl
from jax.experimental.pallas import tpu as pltpu

_DN_T = (((1,), (1,)), ((), ()))

_TB = 2048


def _mlp_kernel(x_ref, w1_ref, b1_ref, w2_ref, b2_ref, out_ref):
    h = jax.lax.dot_general(x_ref[...], w1_ref[...], _DN_T,
                            preferred_element_type=jnp.float32)
    h = jnp.maximum(h + b1_ref[...], 0.0)
    o = jax.lax.dot_general(h, w2_ref[...], _DN_T,
                            preferred_element_type=jnp.float32)
    out_ref[...] = o + b2_ref[...]


def kernel(x, w1, b1, w2, b2):
    B, F = x.shape
    H = w1.shape[0]
    O = w2.shape[0]

    b1r = b1.reshape(1, H)
    b2r = b2.reshape(1, O)

    return pl.pallas_call(
        _mlp_kernel,
        out_shape=jax.ShapeDtypeStruct((B, O), jnp.float32),
        grid=(pl.cdiv(B, _TB),),
        in_specs=[
            pl.BlockSpec((_TB, F), lambda i: (i, 0)),
            pl.BlockSpec((H, F), lambda i: (0, 0)),
            pl.BlockSpec((1, H), lambda i: (0, 0)),
            pl.BlockSpec((O, H), lambda i: (0, 0)),
            pl.BlockSpec((1, O), lambda i: (0, 0)),
        ],
        out_specs=pl.BlockSpec((_TB, O), lambda i: (i, 0)),
        compiler_params=pltpu.CompilerParams(
            dimension_semantics=("arbitrary",),
        ),
        cost_estimate=pl.CostEstimate(
            flops=2 * B * (F * H + H * O),
            transcendentals=0,
            bytes_accessed=4 * (B * F + B * O + F * H + H * O),
        ),
    )(x, w1, b1r, w2, b2r)


# speedup vs baseline: 1.0104x; 1.0072x over previous
import jax
import jax.numpy as jnp
from jax.experimental import pallas as pl
from jax.experimental.pallas import tpu as pltpu

_DN_T = (((1,), (1,)), ((), ()))

_TB = 1024


def _mlp_kernel(x_ref, w1_ref, b1_ref, w2_hbm_ref, b2_ref, out_ref,
                w2_vmem, w2_sem):
    i = pl.program_id(0)
    w2_copy = pltpu.make_async_copy(w2_hbm_ref, w2_vmem, w2_sem)

    @pl.when(i == 0)
    def _():
        w2_copy.start()

    h = jax.lax.dot_general(x_ref[...], w1_ref[...], _DN_T,
                            preferred_element_type=jnp.float32)
    h = jnp.maximum(h + b1_ref[...], 0.0)

    @pl.when(i == 0)
    def _():
        w2_copy.wait()

    o = jax.lax.dot_general(h, w2_vmem[...], _DN_T,
                            preferred_element_type=jnp.float32)
    out_ref[...] = o + b2_ref[...]


def kernel(x, w1, b1, w2, b2):
    B, F = x.shape
    H = w1.shape[0]
    O = w2.shape[0]

    b1r = b1.reshape(1, H)
    b2r = b2.reshape(1, O)

    return pl.pallas_call(
        _mlp_kernel,
        out_shape=jax.ShapeDtypeStruct((B, O), jnp.float32),
        grid=(pl.cdiv(B, _TB),),
        in_specs=[
            pl.BlockSpec((_TB, F), lambda i: (i, 0)),
            pl.BlockSpec((H, F), lambda i: (0, 0)),
            pl.BlockSpec((1, H), lambda i: (0, 0)),
            pl.BlockSpec(memory_space=pltpu.MemorySpace.HBM),
            pl.BlockSpec((1, O), lambda i: (0, 0)),
        ],
        out_specs=pl.BlockSpec((_TB, O), lambda i: (i, 0)),
        scratch_shapes=[
            pltpu.VMEM((O, H), jnp.float32),
            pltpu.SemaphoreType.DMA,
        ],
        compiler_params=pltpu.CompilerParams(
            dimension_semantics=("arbitrary",),
        ),
        cost_estimate=pl.CostEstimate(
            flops=2 * B * (F * H + H * O),
            transcendentals=0,
            bytes_accessed=4 * (B * F + B * O + F * H + H * O),
        ),
    )(x, w1, b1r, w2, b2r)


# speedup vs baseline: 1.0247x; 1.0142x over previous
import jax
import jax.numpy as jnp
from jax.experimental import pallas as pl
from jax.experimental.pallas import tpu as pltpu

_DN_T = (((1,), (1,)), ((), ()))

_TB = 1024


def _mlp_kernel(x_ref, w1_ref, b1_ref, w2_ref, b2_ref, out_ref):
    h = jax.lax.dot_general(x_ref[...], w1_ref[...], _DN_T,
                            preferred_element_type=jnp.float32)
    h = jnp.maximum(h + b1_ref[...], 0.0)
    o = jax.lax.dot_general(h, w2_ref[...], _DN_T,
                            preferred_element_type=jnp.float32)
    out_ref[...] = o + b2_ref[...]


def kernel(x, w1, b1, w2, b2):
    B, F = x.shape
    H = w1.shape[0]
    O = w2.shape[0]

    b1r = b1.reshape(1, H)
    b2r = b2.reshape(1, O)

    return pl.pallas_call(
        _mlp_kernel,
        out_shape=jax.ShapeDtypeStruct((B, O), jnp.float32),
        grid=(pl.cdiv(B, _TB),),
        in_specs=[
            pl.BlockSpec((_TB, F), lambda i: (i, 0)),
            pl.BlockSpec((H, F), lambda i: (0, 0)),
            pl.BlockSpec((1, H), lambda i: (0, 0)),
            pl.BlockSpec((O, H), lambda i: (0, 0)),
            pl.BlockSpec((1, O), lambda i: (0, 0)),
        ],
        out_specs=pl.BlockSpec((_TB, O), lambda i: (i, 0)),
        compiler_params=pltpu.CompilerParams(
            dimension_semantics=("arbitrary",),
        ),
        cost_estimate=pl.CostEstimate(
            flops=2 * B * (F * H + H * O),
            transcendentals=0,
            bytes_accessed=4 * (B * F + B * O + F * H + H * O),
        ),
    )(x, w1, b1r, w2, b2r)


# speedup vs baseline: 1.0266x; 1.0019x over previous
import jax
import jax.numpy as jnp
from jax.experimental import pallas as pl
from jax.experimental.pallas import tpu as pltpu

_DN_T = (((1,), (1,)), ((), ()))

_TB = 1024


_CHUNK = 512


def _mlp_kernel(x_ref, w1_ref, b1_ref, w2_ref, b2_ref, out_ref):
    w1 = w1_ref[...]
    w2 = w2_ref[...]
    b1 = b1_ref[...]
    b2 = b2_ref[...]
    for r in range(0, _TB, _CHUNK):
        h = jax.lax.dot_general(x_ref[r:r + _CHUNK, :], w1, _DN_T,
                                preferred_element_type=jnp.float32)
        h = jnp.maximum(h + b1, 0.0)
        o = jax.lax.dot_general(h, w2, _DN_T,
                                preferred_element_type=jnp.float32)
        out_ref[r:r + _CHUNK, :] = o + b2


def kernel(x, w1, b1, w2, b2):
    B, F = x.shape
    H = w1.shape[0]
    O = w2.shape[0]

    b1r = b1.reshape(1, H)
    b2r = b2.reshape(1, O)

    return pl.pallas_call(
        _mlp_kernel,
        out_shape=jax.ShapeDtypeStruct((B, O), jnp.float32),
        grid=(pl.cdiv(B, _TB),),
        in_specs=[
            pl.BlockSpec((_TB, F), lambda i: (i, 0)),
            pl.BlockSpec((H, F), lambda i: (0, 0)),
            pl.BlockSpec((1, H), lambda i: (0, 0)),
            pl.BlockSpec((O, H), lambda i: (0, 0)),
            pl.BlockSpec((1, O), lambda i: (0, 0)),
        ],
        out_specs=pl.BlockSpec((_TB, O), lambda i: (i, 0)),
        compiler_params=pltpu.CompilerParams(
            dimension_semantics=("arbitrary",),
        ),
        cost_estimate=pl.CostEstimate(
            flops=2 * B * (F * H + H * O),
            transcendentals=0,
            bytes_accessed=4 * (B * F + B * O + F * H + H * O),
        ),
    )(x, w1, b1r, w2, b2r)
